# baseline (speedup 1.0000x reference)
import functools

import jax
import jax.numpy as jnp
from jax import lax
from jax.experimental import pallas as pl
from jax.experimental.pallas import tpu as pltpu
from jax.experimental.pallas import tpu_sc as plsc

BATCH = 16384
EMBED_DIM = 32
NUM_ROWS = 1000000
NUM_CORES = 2
NUM_SUBCORES = 16
LANES = 16
NUM_WORKERS = NUM_CORES * NUM_SUBCORES
B_PER_W = BATCH // NUM_WORKERS
CHUNK = 256
ROWS_PER_ENTRY = 4
SLAB = 128
NUM_SLABS = -(-NUM_ROWS // SLAB)
NUM_ENTRIES = NUM_SLABS * SLAB // ROWS_PER_ENTRY
MAIN_SLABS = 244 * NUM_WORKERS
ENT_PER_SLAB = SLAB // ROWS_PER_ENTRY


def _transpose_body(ut_hbm, it_hbm, ou_hbm, oi_hbm,
                    bufs_u, bufs_i, out_u, out_i, sem_in, sem_out):
    wid = lax.axis_index("s") * NUM_CORES + lax.axis_index("c")

    lane = lax.iota(jnp.int32, LANES)
    d_oct = []
    c_oct = []
    for oct_ in range(8):
        k = oct_ * LANES + lane
        q = jax.lax.shift_right_logical(k, 5)
        d = jax.lax.bitwise_and(k, 31)
        d_oct.append(d)
        c_oct.append(q)

    def start_reads(j, par):
        for i in range(4):
            pltpu.async_copy(
                ut_hbm.at[pl.ds(8 * i, 8), pl.ds(SLAB * j, SLAB)],
                bufs_u.at[par].at[pl.ds(8 * i, 8), :], sem_in)
            pltpu.async_copy(
                it_hbm.at[pl.ds(8 * i, 8), pl.ds(SLAB * j, SLAB)],
                bufs_i.at[par].at[pl.ds(8 * i, 8), :], sem_in)

    def wait_reads(par):
        for i in range(4):
            pltpu.make_async_copy(
                ut_hbm.at[pl.ds(0, 8), pl.ds(0, SLAB)],
                bufs_u.at[par].at[pl.ds(8 * i, 8), :], sem_in).wait()
            pltpu.make_async_copy(
                it_hbm.at[pl.ds(0, 8), pl.ds(0, SLAB)],
                bufs_i.at[par].at[pl.ds(8 * i, 8), :], sem_in).wait()

    def extract(par):
        def one(m, carry):
            col = 4 * m
            for oct_ in range(8):
                u = plsc.load_gather(bufs_u.at[par], [d_oct[oct_],
                                                      c_oct[oct_] + col])
                v = plsc.load_gather(bufs_i.at[par], [d_oct[oct_],
                                                      c_oct[oct_] + col])
                out_u[m, pl.ds(oct_ * LANES, LANES)] = u
                out_i[m, pl.ds(oct_ * LANES, LANES)] = v
            return carry
        lax.fori_loop(0, ENT_PER_SLAB, one, 0)

    def fire_writes(j):
        pltpu.async_copy(out_u, ou_hbm.at[pl.ds(ENT_PER_SLAB * j, ENT_PER_SLAB)],
                         sem_out)
        pltpu.async_copy(out_i, oi_hbm.at[pl.ds(ENT_PER_SLAB * j, ENT_PER_SLAB)],
                         sem_out)

    def wait_writes(j_prev):
        pltpu.make_async_copy(
            out_u, ou_hbm.at[pl.ds(ENT_PER_SLAB * j_prev, ENT_PER_SLAB)],
            sem_out).wait()
        pltpu.make_async_copy(
            out_i, oi_hbm.at[pl.ds(ENT_PER_SLAB * j_prev, ENT_PER_SLAB)],
            sem_out).wait()

    start_reads(wid, 0)

    def pair(kk, carry):
        k0 = 2 * kk
        k1 = 2 * kk + 1

        @pl.when(k1 < 244)
        def _():
            start_reads(wid + NUM_WORKERS * k1, 1)
        wait_reads(0)

        @pl.when(kk > 0)
        def _():
            wait_writes(wid + NUM_WORKERS * (k0 - 1))
        extract(0)
        fire_writes(wid + NUM_WORKERS * k0)

        @pl.when(k1 + 1 < 244)
        def _():
            start_reads(wid + NUM_WORKERS * (k1 + 1), 0)
        wait_reads(1)
        wait_writes(wid + NUM_WORKERS * k0)
        extract(1)
        fire_writes(wid + NUM_WORKERS * k1)
        return carry

    lax.fori_loop(0, 122, pair, 0)
    wait_writes(wid + NUM_WORKERS * 243)

    def tail_slab(j, width, n_ent):
        for i in range(4):
            pltpu.async_copy(
                ut_hbm.at[pl.ds(8 * i, 8), pl.ds(SLAB * j, width)],
                bufs_u.at[0].at[pl.ds(8 * i, 8), pl.ds(0, width)], sem_in)
            pltpu.async_copy(
                it_hbm.at[pl.ds(8 * i, 8), pl.ds(SLAB * j, width)],
                bufs_i.at[0].at[pl.ds(8 * i, 8), pl.ds(0, width)], sem_in)
        for i in range(4):
            pltpu.make_async_copy(
                ut_hbm.at[pl.ds(0, 8), pl.ds(0, width)],
                bufs_u.at[0].at[pl.ds(8 * i, 8), pl.ds(0, width)],
                sem_in).wait()
            pltpu.make_async_copy(
                it_hbm.at[pl.ds(0, 8), pl.ds(0, width)],
                bufs_i.at[0].at[pl.ds(8 * i, 8), pl.ds(0, width)],
                sem_in).wait()

        def one(m, carry):
            col = 4 * m
            for oct_ in range(8):
                u = plsc.load_gather(bufs_u.at[0], [d_oct[oct_],
                                                    c_oct[oct_] + col])
                v = plsc.load_gather(bufs_i.at[0], [d_oct[oct_],
                                                    c_oct[oct_] + col])
                out_u[m, pl.ds(oct_ * LANES, LANES)] = u
                out_i[m, pl.ds(oct_ * LANES, LANES)] = v
            return carry
        lax.fori_loop(0, n_ent, one, 0)
        pltpu.sync_copy(out_u.at[pl.ds(0, n_ent)],
                        ou_hbm.at[pl.ds(ENT_PER_SLAB * j, n_ent)])
        pltpu.sync_copy(out_i.at[pl.ds(0, n_ent)],
                        oi_hbm.at[pl.ds(ENT_PER_SLAB * j, n_ent)])

    @pl.when(wid < 5)
    def _():
        tail_slab(7808 + wid, SLAB, ENT_PER_SLAB)


def _gather_dot_body(uid_hbm, iid_hbm, ut_hbm, it_hbm, out_hbm,
                     uid_v, iid_v, uent_v, ient_v, ubuf, ibuf, out_v,
                     sem_u, sem_i):
    wid = lax.axis_index("s") * NUM_CORES + lax.axis_index("c")
    base = wid * B_PER_W

    pltpu.sync_copy(uid_hbm.at[pl.ds(base, B_PER_W)], uid_v)
    pltpu.sync_copy(iid_hbm.at[pl.ds(base, B_PER_W)], iid_v)

    def split(g, carry):
        c0 = g * LANES
        u = uid_v[pl.ds(c0, LANES)]
        i = iid_v[pl.ds(c0, LANES)]
        uent_v[pl.ds(c0, LANES)] = jax.lax.shift_right_logical(u, 2)
        ient_v[pl.ds(c0, LANES)] = jax.lax.shift_right_logical(i, 2)
        return carry
    lax.fori_loop(0, B_PER_W // LANES, split, 0)

    lane = lax.iota(jnp.int32, LANES)

    def chunk(h, carry):
        c0 = h * CHUNK
        cp_u = pltpu.async_copy(ut_hbm.at[uent_v.at[pl.ds(c0, CHUNK)]],
                                ubuf, sem_u)
        cp_i = pltpu.async_copy(it_hbm.at[ient_v.at[pl.ds(c0, CHUNK)]],
                                ibuf, sem_i)
        cp_u.wait()
        cp_i.wait()

        def group(g, carry2):
            gc = g * LANES
            cv = gc + lane
            uq = jax.lax.rem(uid_v[pl.ds(c0 + gc, LANES)], 4) * EMBED_DIM
            iq = jax.lax.rem(iid_v[pl.ds(c0 + gc, LANES)], 4) * EMBED_DIM
            acc = jnp.zeros((LANES,), jnp.float32)
            for j in range(EMBED_DIM):
                u = plsc.load_gather(ubuf, [cv, uq + j])
                v = plsc.load_gather(ibuf, [cv, iq + j])
                acc = acc + u * v
            out_v[pl.ds(c0 + gc, LANES)] = acc
            return carry2
        lax.fori_loop(0, CHUNK // LANES, group, 0)
        return carry
    lax.fori_loop(0, B_PER_W // CHUNK, chunk, 0)

    pltpu.sync_copy(out_v, out_hbm.at[pl.ds(base, B_PER_W)])


@jax.jit
def kernel(user_ids, item_ids, user_table, item_table):
    mesh = plsc.VectorSubcoreMesh(
        core_axis_name="c", subcore_axis_name="s",
        num_cores=NUM_CORES, num_subcores=NUM_SUBCORES)

    trans = pl.kernel(
        _transpose_body,
        out_type=(jax.ShapeDtypeStruct((NUM_ENTRIES, 128), jnp.float32),
                  jax.ShapeDtypeStruct((NUM_ENTRIES, 128), jnp.float32)),
        mesh=mesh,
        compiler_params=pltpu.CompilerParams(needs_layout_passes=False),
        scratch_types=[
            pltpu.VMEM((2, EMBED_DIM, SLAB), jnp.float32),
            pltpu.VMEM((2, EMBED_DIM, SLAB), jnp.float32),
            pltpu.VMEM((ENT_PER_SLAB, 128), jnp.float32),
            pltpu.VMEM((ENT_PER_SLAB, 128), jnp.float32),
            pltpu.SemaphoreType.DMA,
            pltpu.SemaphoreType.DMA,
        ],
    )
    ut_e, it_e = trans(user_table.T, item_table.T)

    gather = pl.kernel(
        _gather_dot_body,
        out_type=jax.ShapeDtypeStruct((BATCH,), jnp.float32),
        mesh=mesh,
        compiler_params=pltpu.CompilerParams(needs_layout_passes=False),
        scratch_types=[
            pltpu.VMEM((B_PER_W,), jnp.int32),
            pltpu.VMEM((B_PER_W,), jnp.int32),
            pltpu.VMEM((B_PER_W,), jnp.int32),
            pltpu.VMEM((B_PER_W,), jnp.int32),
            pltpu.VMEM((CHUNK, 128), jnp.float32),
            pltpu.VMEM((CHUNK, 128), jnp.float32),
            pltpu.VMEM((B_PER_W,), jnp.float32),
            pltpu.SemaphoreType.DMA,
            pltpu.SemaphoreType.DMA,
        ],
    )
    return gather(user_ids.astype(jnp.int32), item_ids.astype(jnp.int32),
                  ut_e, it_e)


# speedup vs baseline: 1.0586x; 1.0586x over previous
import functools

import jax
import jax.numpy as jnp
from jax import lax
from jax.experimental import pallas as pl
from jax.experimental.pallas import tpu as pltpu
from jax.experimental.pallas import tpu_sc as plsc

BATCH = 16384
EMBED_DIM = 32
NUM_ROWS = 1000000
NUM_CORES = 2
NUM_SUBCORES = 16
LANES = 16
NUM_WORKERS = NUM_CORES * NUM_SUBCORES
B_PER_W = BATCH // NUM_WORKERS
CHUNK = 256
ROWS_PER_ENTRY = 4
SLAB = 128
NUM_SLABS = -(-NUM_ROWS // SLAB)
NUM_ENTRIES = NUM_SLABS * SLAB // ROWS_PER_ENTRY
MAIN_SLABS = 244 * NUM_WORKERS
ENT_PER_SLAB = SLAB // ROWS_PER_ENTRY


def _transpose_body(ut_hbm, it_hbm, ou_hbm, oi_hbm,
                    bufs_u, bufs_i, out_u, out_i, sem_in, sem_out):
    wid = lax.axis_index("s") * NUM_CORES + lax.axis_index("c")

    lane = lax.iota(jnp.int32, LANES)
    d_oct = []
    c_oct = []
    for oct_ in range(8):
        k = oct_ * LANES + lane
        q = jax.lax.shift_right_logical(k, 5)
        d = jax.lax.bitwise_and(k, 31)
        d_oct.append(d)
        c_oct.append(q)

    def start_reads(j, par):
        pltpu.async_copy(ut_hbm.at[:, pl.ds(SLAB * j, SLAB)],
                         bufs_u.at[par], sem_in)
        pltpu.async_copy(it_hbm.at[:, pl.ds(SLAB * j, SLAB)],
                         bufs_i.at[par], sem_in)

    def wait_reads(par):
        pltpu.make_async_copy(ut_hbm.at[:, pl.ds(0, SLAB)],
                              bufs_u.at[par], sem_in).wait()
        pltpu.make_async_copy(it_hbm.at[:, pl.ds(0, SLAB)],
                              bufs_i.at[par], sem_in).wait()

    def extract(par):
        def one(m, carry):
            col = 4 * m
            for oct_ in range(8):
                u = plsc.load_gather(bufs_u.at[par], [d_oct[oct_],
                                                      c_oct[oct_] + col])
                v = plsc.load_gather(bufs_i.at[par], [d_oct[oct_],
                                                      c_oct[oct_] + col])
                out_u[par, m, pl.ds(oct_ * LANES, LANES)] = u
                out_i[par, m, pl.ds(oct_ * LANES, LANES)] = v
            return carry
        lax.fori_loop(0, ENT_PER_SLAB, one, 0)

    def fire_writes(j, par):
        pltpu.async_copy(out_u.at[par],
                         ou_hbm.at[pl.ds(ENT_PER_SLAB * j, ENT_PER_SLAB)],
                         sem_out)
        pltpu.async_copy(out_i.at[par],
                         oi_hbm.at[pl.ds(ENT_PER_SLAB * j, ENT_PER_SLAB)],
                         sem_out)

    def wait_writes(j_prev, par):
        pltpu.make_async_copy(
            out_u.at[par],
            ou_hbm.at[pl.ds(ENT_PER_SLAB * j_prev, ENT_PER_SLAB)],
            sem_out).wait()
        pltpu.make_async_copy(
            out_i.at[par],
            oi_hbm.at[pl.ds(ENT_PER_SLAB * j_prev, ENT_PER_SLAB)],
            sem_out).wait()

    start_reads(wid, 0)

    def pair(kk, carry):
        k0 = 2 * kk
        k1 = 2 * kk + 1

        @pl.when(k1 < 244)
        def _():
            start_reads(wid + NUM_WORKERS * k1, 1)
        wait_reads(0)

        @pl.when(kk > 0)
        def _():
            wait_writes(wid + NUM_WORKERS * (k0 - 2), 0)
        extract(0)
        fire_writes(wid + NUM_WORKERS * k0, 0)

        @pl.when(k1 + 1 < 244)
        def _():
            start_reads(wid + NUM_WORKERS * (k1 + 1), 0)
        wait_reads(1)

        @pl.when(kk > 0)
        def _():
            wait_writes(wid + NUM_WORKERS * (k1 - 2), 1)
        extract(1)
        fire_writes(wid + NUM_WORKERS * k1, 1)
        return carry

    lax.fori_loop(0, 122, pair, 0)
    wait_writes(wid + NUM_WORKERS * 242, 0)
    wait_writes(wid + NUM_WORKERS * 243, 1)

    def tail_slab(j, n_ent):
        pltpu.sync_copy(ut_hbm.at[:, pl.ds(SLAB * j, SLAB)], bufs_u.at[0])
        pltpu.sync_copy(it_hbm.at[:, pl.ds(SLAB * j, SLAB)], bufs_i.at[0])

        def one(m, carry):
            col = 4 * m
            for oct_ in range(8):
                u = plsc.load_gather(bufs_u.at[0], [d_oct[oct_],
                                                    c_oct[oct_] + col])
                v = plsc.load_gather(bufs_i.at[0], [d_oct[oct_],
                                                    c_oct[oct_] + col])
                out_u[0, m, pl.ds(oct_ * LANES, LANES)] = u
                out_i[0, m, pl.ds(oct_ * LANES, LANES)] = v
            return carry
        lax.fori_loop(0, n_ent, one, 0)
        pltpu.sync_copy(out_u.at[0],
                        ou_hbm.at[pl.ds(ENT_PER_SLAB * j, ENT_PER_SLAB)])
        pltpu.sync_copy(out_i.at[0],
                        oi_hbm.at[pl.ds(ENT_PER_SLAB * j, ENT_PER_SLAB)])

    @pl.when(wid < 5)
    def _():
        tail_slab(7808 + wid, ENT_PER_SLAB)


def _gather_dot_body(uid_hbm, iid_hbm, ut_hbm, it_hbm, out_hbm,
                     uid_v, iid_v, uent_v, ient_v, ubuf, ibuf, out_v,
                     sem_u, sem_i):
    wid = lax.axis_index("s") * NUM_CORES + lax.axis_index("c")
    base = wid * B_PER_W

    pltpu.sync_copy(uid_hbm.at[pl.ds(base, B_PER_W)], uid_v)
    pltpu.sync_copy(iid_hbm.at[pl.ds(base, B_PER_W)], iid_v)

    def split(g, carry):
        c0 = g * LANES
        u = uid_v[pl.ds(c0, LANES)]
        i = iid_v[pl.ds(c0, LANES)]
        uent_v[pl.ds(c0, LANES)] = jax.lax.shift_right_logical(u, 2)
        ient_v[pl.ds(c0, LANES)] = jax.lax.shift_right_logical(i, 2)
        return carry
    lax.fori_loop(0, B_PER_W // LANES, split, 0)

    lane = lax.iota(jnp.int32, LANES)

    def chunk(h, carry):
        c0 = h * CHUNK
        cp_u = pltpu.async_copy(ut_hbm.at[uent_v.at[pl.ds(c0, CHUNK)]],
                                ubuf, sem_u)
        cp_i = pltpu.async_copy(it_hbm.at[ient_v.at[pl.ds(c0, CHUNK)]],
                                ibuf, sem_i)
        cp_u.wait()
        cp_i.wait()

        def group(g, carry2):
            gc = g * LANES
            cv = gc + lane
            uq = jax.lax.rem(uid_v[pl.ds(c0 + gc, LANES)], 4) * EMBED_DIM
            iq = jax.lax.rem(iid_v[pl.ds(c0 + gc, LANES)], 4) * EMBED_DIM
            acc = jnp.zeros((LANES,), jnp.float32)
            for j in range(EMBED_DIM):
                u = plsc.load_gather(ubuf, [cv, uq + j])
                v = plsc.load_gather(ibuf, [cv, iq + j])
                acc = acc + u * v
            out_v[pl.ds(c0 + gc, LANES)] = acc
            return carry2
        lax.fori_loop(0, CHUNK // LANES, group, 0)
        return carry
    lax.fori_loop(0, B_PER_W // CHUNK, chunk, 0)

    pltpu.sync_copy(out_v, out_hbm.at[pl.ds(base, B_PER_W)])


@jax.jit
def kernel(user_ids, item_ids, user_table, item_table):
    mesh = plsc.VectorSubcoreMesh(
        core_axis_name="c", subcore_axis_name="s",
        num_cores=NUM_CORES, num_subcores=NUM_SUBCORES)

    trans = pl.kernel(
        _transpose_body,
        out_type=(jax.ShapeDtypeStruct((NUM_ENTRIES, 128), jnp.float32),
                  jax.ShapeDtypeStruct((NUM_ENTRIES, 128), jnp.float32)),
        mesh=mesh,
        compiler_params=pltpu.CompilerParams(needs_layout_passes=False),
        scratch_types=[
            pltpu.VMEM((2, EMBED_DIM, SLAB), jnp.float32),
            pltpu.VMEM((2, EMBED_DIM, SLAB), jnp.float32),
            pltpu.VMEM((2, ENT_PER_SLAB, 128), jnp.float32),
            pltpu.VMEM((2, ENT_PER_SLAB, 128), jnp.float32),
            pltpu.SemaphoreType.DMA,
            pltpu.SemaphoreType.DMA,
        ],
    )
    ut_e, it_e = trans(user_table.T, item_table.T)

    gather = pl.kernel(
        _gather_dot_body,
        out_type=jax.ShapeDtypeStruct((BATCH,), jnp.float32),
        mesh=mesh,
        compiler_params=pltpu.CompilerParams(needs_layout_passes=False),
        scratch_types=[
            pltpu.VMEM((B_PER_W,), jnp.int32),
            pltpu.VMEM((B_PER_W,), jnp.int32),
            pltpu.VMEM((B_PER_W,), jnp.int32),
            pltpu.VMEM((B_PER_W,), jnp.int32),
            pltpu.VMEM((CHUNK, 128), jnp.float32),
            pltpu.VMEM((CHUNK, 128), jnp.float32),
            pltpu.VMEM((B_PER_W,), jnp.float32),
            pltpu.SemaphoreType.DMA,
            pltpu.SemaphoreType.DMA,
        ],
    )
    return gather(user_ids.astype(jnp.int32), item_ids.astype(jnp.int32),
                  ut_e, it_e)


# speedup vs baseline: 1.6378x; 1.5472x over previous
import functools

import jax
import jax.numpy as jnp
from jax import lax
from jax.experimental import pallas as pl
from jax.experimental.pallas import tpu as pltpu
from jax.experimental.pallas import tpu_sc as plsc

BATCH = 16384
EMBED_DIM = 32
NUM_CORES = 2
NUM_SUBCORES = 16
LANES = 16
NUM_WORKERS = NUM_CORES * NUM_SUBCORES
B_PER_W = BATCH // NUM_WORKERS
GROUPS = B_PER_W // LANES


def _body(uid_hbm, iid_hbm, ut_hbm, it_hbm, out_hbm,
          uid_v, iid_v, urows, irows, out_v, sem_u, sem_i):
    wid = lax.axis_index("s") * NUM_CORES + lax.axis_index("c")
    base = wid * B_PER_W

    pltpu.sync_copy(uid_hbm.at[pl.ds(base, B_PER_W)], uid_v)
    pltpu.sync_copy(iid_hbm.at[pl.ds(base, B_PER_W)], iid_v)

    cp_u = pltpu.async_copy(ut_hbm.at[uid_v], urows, sem_u)
    cp_i = pltpu.async_copy(it_hbm.at[iid_v], irows, sem_i)
    cp_u.wait()
    cp_i.wait()

    lane = lax.iota(jnp.int32, LANES)

    def group(g, carry):
        r0 = g * LANES
        acc = jnp.zeros((LANES,), jnp.float32)
        for i in range(LANES):
            r = r0 + i
            u0 = urows[r, pl.ds(0, LANES)]
            u1 = urows[r, pl.ds(LANES, LANES)]
            v0 = irows[r, pl.ds(0, LANES)]
            v1 = irows[r, pl.ds(LANES, LANES)]
            p = u0 * v0 + u1 * v1
            acc = jnp.where(lane == i, jnp.sum(p), acc)
        out_v[pl.ds(r0, LANES)] = acc
        return carry

    lax.fori_loop(0, GROUPS, group, 0)

    pltpu.sync_copy(out_v, out_hbm.at[pl.ds(base, B_PER_W)])


@jax.jit
def kernel(user_ids, item_ids, user_table, item_table):
    mesh = plsc.VectorSubcoreMesh(
        core_axis_name="c", subcore_axis_name="s",
        num_cores=NUM_CORES, num_subcores=NUM_SUBCORES)
    f = pl.kernel(
        _body,
        out_type=jax.ShapeDtypeStruct((BATCH,), jnp.float32),
        mesh=mesh,
        compiler_params=pltpu.CompilerParams(
            needs_layout_passes=False, use_tc_tiling_on_sc=False),
        scratch_types=[
            pltpu.VMEM((B_PER_W,), jnp.int32),
            pltpu.VMEM((B_PER_W,), jnp.int32),
            pltpu.VMEM((B_PER_W, EMBED_DIM), jnp.float32),
            pltpu.VMEM((B_PER_W, EMBED_DIM), jnp.float32),
            pltpu.VMEM((B_PER_W,), jnp.float32),
            pltpu.SemaphoreType.DMA,
            pltpu.SemaphoreType.DMA,
        ],
    )
    return f(user_ids.astype(jnp.int32), item_ids.astype(jnp.int32),
             user_table, item_table)
